# baseline (speedup 1.0000x reference)
import jax
import jax.numpy as jnp
from jax.experimental import pallas as pl

_B = 64
_D = 1024
_K = 16
_BEAM = 8
_DEPTH = 4
_NEG = -3.0e38


def _fit_cols(th, wfb, bf, s_count):
    cols = [None] * (_K * s_count)
    for k in range(_K):
        ck = (th + 0.1 * k).astype(jnp.bfloat16)
        fk = jnp.dot(ck, wfb, preferred_element_type=jnp.float32) + bf
        for s in range(s_count):
            cols[k * s_count + s] = fk[s * _B:(s + 1) * _B, :]
    return jnp.concatenate(cols, axis=1)


def _topk_select(fit, t_list, c_iota):
    s_count = len(t_list)
    C = _K * s_count
    out = []
    fits = []
    for _ in range(_BEAM):
        mx = jnp.max(fit, axis=1, keepdims=True)
        cidx = jnp.min(jnp.where(fit == mx, c_iota, C), axis=1,
                       keepdims=True)
        fit = jnp.where(c_iota == cidx, _NEG, fit)
        fits.append(mx)
        k_sel = cidx // s_count
        if s_count == 1:
            row = t_list[0] + 0.1 * k_sel.astype(jnp.float32)
        else:
            s_sel = cidx % s_count
            b0 = (s_sel & 1) > 0
            b1 = (s_sel & 2) > 0
            b2 = (s_sel & 4) > 0
            e = jnp.where(b1, jnp.where(b0, t_list[3], t_list[2]),
                          jnp.where(b0, t_list[1], t_list[0]))
            f = jnp.where(b1, jnp.where(b0, t_list[7], t_list[6]),
                          jnp.where(b0, t_list[5], t_list[4]))
            row = jnp.where(b2, f, e) + 0.1 * k_sel.astype(jnp.float32)
        out.append(row)
    return out, fits


def _search_kernel(x_ref, wt_ref, bt_ref, wf_ref, bf_ref, y_ref):
    wt = wt_ref[...]
    bt = bt_ref[...]
    wfb = wf_ref[...]
    bf = bf_ref[0, 0]
    iota_k = jax.lax.broadcasted_iota(jnp.int32, (_B, _K), 1)
    iota_c = jax.lax.broadcasted_iota(jnp.int32, (_B, _K * _BEAM), 1)

    x = x_ref[...]
    t = jnp.tanh(jnp.dot(x, wt, preferred_element_type=jnp.float32) + bt)
    states, fits = _topk_select(_fit_cols(t, wfb, bf, 1), [t], iota_k)

    for _ in range(_DEPTH - 1):
        flat = jnp.concatenate(states, axis=0)
        th = jnp.tanh(jnp.dot(flat, wt, preferred_element_type=jnp.float32)
                      + bt)
        t_list = [th[s * _B:(s + 1) * _B, :] for s in range(_BEAM)]
        states, fits = _topk_select(_fit_cols(th, wfb, bf, _BEAM), t_list,
                                    iota_c)

    ffit = jnp.concatenate(fits, axis=1)
    w = jax.nn.softmax(ffit, axis=1)
    acc = [states[j] * w[:, j:j + 1] for j in range(_BEAM)]
    y01, y23 = acc[0] + acc[1], acc[2] + acc[3]
    y45, y67 = acc[4] + acc[5], acc[6] + acc[7]
    y_ref[...] = (y01 + y23) + (y45 + y67)


def kernel(x, Wt, bt, wf, bf):
    return pl.pallas_call(
        _search_kernel,
        out_shape=jax.ShapeDtypeStruct((_B, _D), jnp.float32),
    )(x.astype(jnp.bfloat16), Wt.astype(jnp.bfloat16), bt.reshape(1, _D),
      wf.reshape(_D, 1).astype(jnp.bfloat16), bf.reshape(1, 1))


# speedup vs baseline: 1.1078x; 1.1078x over previous
import jax
import jax.numpy as jnp
from jax.experimental import pallas as pl

_B = 64
_D = 1024
_K = 16
_BEAM = 8
_DEPTH = 4
_NEG = -3.0e38


def _fit_cols_blocks(t_list, wfb, bf):
    s_count = len(t_list)
    cols = [None] * (_K * s_count)
    for s in range(s_count):
        ts = t_list[s]
        for k in range(_K):
            ck = (ts + 0.1 * k).astype(jnp.bfloat16)
            cols[k * s_count + s] = jnp.dot(
                ck, wfb, preferred_element_type=jnp.float32) + bf
    return jnp.concatenate(cols, axis=1)


def _topk_select(fit, t_list, c_iota):
    s_count = len(t_list)
    C = _K * s_count
    out = []
    fits = []
    for _ in range(_BEAM):
        mx = jnp.max(fit, axis=1, keepdims=True)
        cidx = jnp.min(jnp.where(fit == mx, c_iota, C), axis=1,
                       keepdims=True)
        fit = jnp.where(c_iota == cidx, _NEG, fit)
        fits.append(mx)
        k_sel = cidx // s_count
        if s_count == 1:
            row = t_list[0] + 0.1 * k_sel.astype(jnp.float32)
        else:
            s_sel = cidx % s_count
            b0 = (s_sel & 1) > 0
            b1 = (s_sel & 2) > 0
            b2 = (s_sel & 4) > 0
            e = jnp.where(b1, jnp.where(b0, t_list[3], t_list[2]),
                          jnp.where(b0, t_list[1], t_list[0]))
            f = jnp.where(b1, jnp.where(b0, t_list[7], t_list[6]),
                          jnp.where(b0, t_list[5], t_list[4]))
            row = jnp.where(b2, f, e) + 0.1 * k_sel.astype(jnp.float32)
        out.append(row)
    return out, fits


def _search_kernel(x_ref, wt_ref, bt_ref, wf_ref, bf_ref, y_ref):
    wt = wt_ref[...]
    bt = bt_ref[...]
    wfb = wf_ref[...].astype(jnp.bfloat16)
    bf = bf_ref[0, 0]
    iota_k = jax.lax.broadcasted_iota(jnp.int32, (_B, _K), 1)
    iota_c = jax.lax.broadcasted_iota(jnp.int32, (_B, _K * _BEAM), 1)

    x = x_ref[...]
    t = jnp.tanh(jnp.dot(x, wt, preferred_element_type=jnp.float32) + bt)
    states, fits = _topk_select(_fit_cols_blocks([t], wfb, bf), [t], iota_k)

    for _ in range(_DEPTH - 1):
        t_list = [jnp.tanh(jnp.dot(sj, wt, preferred_element_type=jnp.float32)
                           + bt) for sj in states]
        states, fits = _topk_select(_fit_cols_blocks(t_list, wfb, bf), t_list,
                                    iota_c)

    ffit = jnp.concatenate(fits, axis=1)
    w = jax.nn.softmax(ffit, axis=1)
    acc = [states[j] * w[:, j:j + 1] for j in range(_BEAM)]
    y01, y23 = acc[0] + acc[1], acc[2] + acc[3]
    y45, y67 = acc[4] + acc[5], acc[6] + acc[7]
    y_ref[...] = (y01 + y23) + (y45 + y67)


def kernel(x, Wt, bt, wf, bf):
    return pl.pallas_call(
        _search_kernel,
        out_shape=jax.ShapeDtypeStruct((_B, _D), jnp.float32),
    )(x, Wt, bt.reshape(1, _D), wf.reshape(_D, 1), bf.reshape(1, 1))


# speedup vs baseline: 1.1427x; 1.0315x over previous
import jax
import jax.numpy as jnp
from jax.experimental import pallas as pl

_B = 64
_D = 1024
_K = 16
_BEAM = 8
_DEPTH = 4
_NEG = -3.0e38


def _fit_cols(th, wfb, bf, s_count):
    cols = [None] * (_K * s_count)
    for k in range(_K):
        ck = (th + 0.1 * k).astype(jnp.bfloat16)
        fk = jnp.dot(ck, wfb, preferred_element_type=jnp.float32) + bf
        for s in range(s_count):
            cols[k * s_count + s] = fk[s * _B:(s + 1) * _B, :]
    return jnp.concatenate(cols, axis=1)


def _topk_select(fit, t_list, c_iota):
    s_count = len(t_list)
    C = _K * s_count
    out = []
    fits = []
    for _ in range(_BEAM):
        mx = jnp.max(fit, axis=1, keepdims=True)
        cidx = jnp.min(jnp.where(fit == mx, c_iota, C), axis=1,
                       keepdims=True)
        fit = jnp.where(c_iota == cidx, _NEG, fit)
        fits.append(mx)
        k_sel = cidx // s_count
        if s_count == 1:
            row = t_list[0] + 0.1 * k_sel.astype(jnp.float32)
        else:
            s_sel = cidx % s_count
            b0 = (s_sel & 1) > 0
            b1 = (s_sel & 2) > 0
            b2 = (s_sel & 4) > 0
            e = jnp.where(b1, jnp.where(b0, t_list[3], t_list[2]),
                          jnp.where(b0, t_list[1], t_list[0]))
            f = jnp.where(b1, jnp.where(b0, t_list[7], t_list[6]),
                          jnp.where(b0, t_list[5], t_list[4]))
            row = jnp.where(b2, f, e) + 0.1 * k_sel.astype(jnp.float32)
        out.append(row)
    return out, fits


def _search_kernel(x_ref, wt_ref, bt_ref, wf_ref, bf_ref, y_ref):
    wt = wt_ref[...]
    bt = bt_ref[...]
    wfb = wf_ref[...].astype(jnp.bfloat16)
    bf = bf_ref[0, 0]
    iota_k = jax.lax.broadcasted_iota(jnp.int32, (_B, _K), 1)
    iota_c = jax.lax.broadcasted_iota(jnp.int32, (_B, _K * _BEAM), 1)

    x = x_ref[...]
    t = jnp.tanh(jnp.dot(x, wt, preferred_element_type=jnp.float32) + bt)
    states, fits = _topk_select(_fit_cols(t, wfb, bf, 1), [t], iota_k)

    for _ in range(_DEPTH - 1):
        flat = jnp.concatenate(states, axis=0)
        th = jnp.tanh(jnp.dot(flat, wt, preferred_element_type=jnp.float32)
                      + bt)
        t_list = [th[s * _B:(s + 1) * _B, :] for s in range(_BEAM)]
        states, fits = _topk_select(_fit_cols(th, wfb, bf, _BEAM), t_list,
                                    iota_c)

    ffit = jnp.concatenate(fits, axis=1)
    w = jax.nn.softmax(ffit, axis=1)
    acc = [states[j] * w[:, j:j + 1] for j in range(_BEAM)]
    y01, y23 = acc[0] + acc[1], acc[2] + acc[3]
    y45, y67 = acc[4] + acc[5], acc[6] + acc[7]
    y_ref[...] = (y01 + y23) + (y45 + y67)


def kernel(x, Wt, bt, wf, bf):
    return pl.pallas_call(
        _search_kernel,
        out_shape=jax.ShapeDtypeStruct((_B, _D), jnp.float32),
    )(x, Wt, bt.reshape(1, _D), wf.reshape(_D, 1), bf.reshape(1, 1))
